# baseline (speedup 1.0000x reference)
import functools

import jax
import jax.numpy as jnp
from jax import lax
from jax.experimental import pallas as pl
from jax.experimental.pallas import tpu as pltpu
from jax.experimental.pallas import tpu_sc as plsc

_B, _D, _R = 16384, 128, 1000
_NC, _NS, _L = 2, 16, 16
_NW = _NC * _NS
_RPW = _B // _NW
_C = 128
_NCHUNK = _RPW // _C


def _build():
    mesh = plsc.VectorSubcoreMesh(core_axis_name="c", subcore_axis_name="s")

    @functools.partial(
        pl.kernel,
        mesh=mesh,
        out_type=jax.ShapeDtypeStruct((_B,), jnp.float32),
        compiler_params=pltpu.CompilerParams(
            needs_layout_passes=False,
            skip_device_barrier=True,
            disable_bounds_checks=True,
            disable_semaphore_checks=True,
        ),
        scratch_types=[
            pltpu.VMEM((_NCHUNK, _C), jnp.int32),
            pltpu.VMEM((3, _C, _D), jnp.float32),
            pltpu.VMEM((2, _C, _D), jnp.float32),
            pltpu.VMEM((2, _C, _D), jnp.float32),
            pltpu.VMEM((_C + _L,), jnp.float32),
            pltpu.VMEM((_L,), jnp.float32),
            pltpu.SemaphoreType.DMA((3,)),
            pltpu.SemaphoreType.DMA((2,)),
            pltpu.SemaphoreType.DMA,
        ],
    )
    def k(s_hbm, r_hbm, o_hbm, rel_hbm, scale_hbm, out_hbm,
          idx_v, w_v, s_v, o_v, out_v, scale_v,
          wsem, sosem, psem):
        wid = lax.axis_index("s") * _NC + lax.axis_index("c")
        base = wid * _RPW
        lane = lax.iota(jnp.int32, _L)
        last_lane = lane == (_L - 1)

        def issue_w(c):
            buf = lax.rem(c, 3)
            return pltpu.async_copy(rel_hbm.at[idx_v.at[c]], w_v.at[buf],
                                    wsem.at[buf])

        def issue_so(c):
            buf = lax.rem(c, 2)
            cb = base + c * _C
            return (
                pltpu.async_copy(s_hbm.at[pl.ds(cb, _C), :], s_v.at[buf],
                                 sosem.at[buf]),
                pltpu.async_copy(o_hbm.at[pl.ds(cb, _C), :], o_v.at[buf],
                                 sosem.at[buf]),
            )

        issue_so(jnp.int32(0))
        idx_cp = pltpu.async_copy(
            r_hbm.at[pl.ds(wid * _NCHUNK, _NCHUNK), :], idx_v, psem)
        scale_cp = pltpu.async_copy(scale_hbm, scale_v, psem)
        idx_cp.wait()
        issue_w(jnp.int32(0))
        issue_w(jnp.int32(1))
        scale_cp.wait()
        scale_vec = scale_v[...]

        def chunk(c, carry):
            sobuf = lax.rem(c, 2)
            wbuf = lax.rem(c, 3)
            cb = base + c * _C
            pltpu.make_async_copy(rel_hbm.at[idx_v.at[c]], w_v.at[wbuf],
                                  wsem.at[wbuf]).wait()
            pltpu.make_async_copy(s_hbm.at[pl.ds(cb, _C), :], s_v.at[sobuf],
                                  sosem.at[sobuf]).wait()
            pltpu.make_async_copy(o_hbm.at[pl.ds(cb, _C), :], o_v.at[sobuf],
                                  sosem.at[sobuf]).wait()

            @pl.when(c + 2 < _NCHUNK)
            def _():
                issue_w(c + 2)

            @pl.when(c + 1 < _NCHUNK)
            def _():
                issue_so(c + 1)

            wb, sb, ob = w_v.at[wbuf], s_v.at[sobuf], o_v.at[sobuf]

            @plsc.parallel_loop(0, _C, 1, unroll=1)
            def row(i, wb=wb, sb=sb, ob=ob):
                t = [(sb[i, pl.ds(j * _L, _L)]
                      * wb[i, pl.ds(j * _L, _L)]
                      * ob[i, pl.ds(j * _L, _L)])
                     for j in range(_D // _L)]
                while len(t) > 1:
                    t = [t[2 * m] + t[2 * m + 1] for m in range(len(t) // 2)]
                cum = plsc.cumsum(t[0])
                plsc.store_compressed(out_v.at[pl.ds(i, _L)], cum,
                                      mask=last_lane)

            for jj in range(_C // _L):
                sl = pl.ds(jj * _L, _L)
                out_v[sl] = out_v[sl] * scale_vec
            pltpu.sync_copy(out_v.at[pl.ds(0, _C)], out_hbm.at[pl.ds(cb, _C)])
            return carry

        lax.fori_loop(0, _NCHUNK, chunk, 0)

    return k


_sc_kernel = _build()


def kernel(s, r, o, rel, scale):
    r32 = r.astype(jnp.int32).reshape(_B // _C, _C)
    scale_vec = jnp.full((_L,), scale, dtype=jnp.float32)
    return _sc_kernel(s, r32, o, rel, scale_vec)


# speedup vs baseline: 1.0062x; 1.0062x over previous
import functools

import jax
import jax.numpy as jnp
from jax import lax
from jax.experimental import pallas as pl
from jax.experimental.pallas import tpu as pltpu
from jax.experimental.pallas import tpu_sc as plsc

_B, _D, _R = 16384, 128, 1000
_NC, _NS, _L = 2, 16, 16
_NW = _NC * _NS
_RPW = _B // _NW
_C = 128
_NCHUNK = _RPW // _C


def _build():
    mesh = plsc.VectorSubcoreMesh(core_axis_name="c", subcore_axis_name="s")

    @functools.partial(
        pl.kernel,
        mesh=mesh,
        out_type=jax.ShapeDtypeStruct((_B,), jnp.float32),
        compiler_params=pltpu.CompilerParams(
            needs_layout_passes=False,
            skip_device_barrier=True,
            disable_bounds_checks=True,
            disable_semaphore_checks=True,
        ),
        scratch_types=[
            pltpu.VMEM((_NCHUNK, _C), jnp.int32),
            pltpu.VMEM((3, _C, _D), jnp.float32),
            pltpu.VMEM((2, _C, _D), jnp.float32),
            pltpu.VMEM((2, _C, _D), jnp.float32),
            pltpu.VMEM((_C + _L,), jnp.float32),
            pltpu.VMEM((_L,), jnp.float32),
            pltpu.SemaphoreType.DMA((3,)),
            pltpu.SemaphoreType.DMA((2,)),
            pltpu.SemaphoreType.DMA,
        ],
    )
    def k(s_hbm, r_hbm, o_hbm, rel_hbm, scale_hbm, out_hbm,
          idx_v, w_v, s_v, o_v, out_v, scale_v,
          wsem, sosem, psem):
        wid = lax.axis_index("s") * _NC + lax.axis_index("c")
        base = wid * _RPW
        lane = lax.iota(jnp.int32, _L)
        last_lane = lane == (_L - 1)

        def issue_w(c):
            buf = lax.rem(c, 3)
            return pltpu.async_copy(rel_hbm.at[idx_v.at[c]], w_v.at[buf],
                                    wsem.at[buf])

        def issue_so(c):
            buf = lax.rem(c, 2)
            cb = base + c * _C
            return (
                pltpu.async_copy(s_hbm.at[pl.ds(cb, _C), :], s_v.at[buf],
                                 sosem.at[buf]),
                pltpu.async_copy(o_hbm.at[pl.ds(cb, _C), :], o_v.at[buf],
                                 sosem.at[buf]),
            )

        issue_so(jnp.int32(0))
        idx_cp = pltpu.async_copy(
            r_hbm.at[pl.ds(wid * _NCHUNK, _NCHUNK), :], idx_v, psem)
        scale_cp = pltpu.async_copy(scale_hbm, scale_v, psem)
        idx_cp.wait()
        issue_w(jnp.int32(0))
        issue_w(jnp.int32(1))
        scale_cp.wait()
        scale_vec = scale_v[...]

        def chunk(c, carry):
            sobuf = lax.rem(c, 2)
            wbuf = lax.rem(c, 3)
            cb = base + c * _C
            pltpu.make_async_copy(rel_hbm.at[idx_v.at[c]], w_v.at[wbuf],
                                  wsem.at[wbuf]).wait()
            pltpu.make_async_copy(s_hbm.at[pl.ds(cb, _C), :], s_v.at[sobuf],
                                  sosem.at[sobuf]).wait()
            pltpu.make_async_copy(o_hbm.at[pl.ds(cb, _C), :], o_v.at[sobuf],
                                  sosem.at[sobuf]).wait()

            @pl.when(c + 2 < _NCHUNK)
            def _():
                issue_w(c + 2)

            @pl.when(c + 1 < _NCHUNK)
            def _():
                issue_so(c + 1)

            wb, sb, ob = w_v.at[wbuf], s_v.at[sobuf], o_v.at[sobuf]

            @plsc.parallel_loop(0, _C, 1, unroll=2)
            def row(i, wb=wb, sb=sb, ob=ob):
                t = [(sb[i, pl.ds(j * _L, _L)]
                      * wb[i, pl.ds(j * _L, _L)]
                      * ob[i, pl.ds(j * _L, _L)])
                     for j in range(_D // _L)]
                while len(t) > 1:
                    t = [t[2 * m] + t[2 * m + 1] for m in range(len(t) // 2)]
                cum = plsc.cumsum(t[0])
                plsc.store_compressed(out_v.at[pl.ds(i, _L)], cum,
                                      mask=last_lane)

            for jj in range(_C // _L):
                sl = pl.ds(jj * _L, _L)
                out_v[sl] = out_v[sl] * scale_vec
            pltpu.sync_copy(out_v.at[pl.ds(0, _C)], out_hbm.at[pl.ds(cb, _C)])
            return carry

        lax.fori_loop(0, _NCHUNK, chunk, 0)

    return k


_sc_kernel = _build()


def kernel(s, r, o, rel, scale):
    r32 = r.astype(jnp.int32).reshape(_B // _C, _C)
    scale_vec = jnp.full((_L,), scale, dtype=jnp.float32)
    return _sc_kernel(s, r32, o, rel, scale_vec)
